# baseline (speedup 1.0000x reference)
import jax
import jax.numpy as jnp
from jax import lax
from jax.experimental import pallas as pl
from jax.experimental.pallas import tpu as pltpu
from jax.experimental.pallas import tpu_sc as plsc

NC = 2
NS = 16
NW = NC * NS
DIM = 64
SCALE = 8.0
NBUF = 4

B, S = 4096, 200
RPW = B // NW
NSTEP = RPW // NBUF


def _body(x_hbm, table_hbm, out_hbm, idx_v, gbuf, sbuf, sem_g, sem_s):
    wid = lax.axis_index("s") * NC + lax.axis_index("c")
    base = wid * RPW
    pltpu.sync_copy(x_hbm.at[pl.ds(base, RPW)], idx_v)

    def start_gather(j, b):
        pltpu.async_copy(table_hbm.at[idx_v.at[j]], gbuf.at[b], sem_g.at[b])

    def wait_gather(j, b):
        pltpu.make_async_copy(table_hbm.at[idx_v.at[j]], gbuf.at[b],
                              sem_g.at[b]).wait()

    def start_store(j, b):
        pltpu.async_copy(sbuf.at[b], out_hbm.at[base + j], sem_s.at[b])

    def wait_store(j, b):
        pltpu.make_async_copy(sbuf.at[b], out_hbm.at[base + j],
                              sem_s.at[b]).wait()

    def scale(b):
        def srow(r2, _):
            for dr in range(2):
                for k in range(DIM // 16):
                    sl = pl.ds(16 * k, 16)
                    sbuf[b, 2 * r2 + dr, sl] = gbuf[b, 2 * r2 + dr, sl] * SCALE
            return 0
        lax.fori_loop(0, S // 2, srow, 0)

    for b in range(NBUF):
        start_gather(b, b)

    for b in range(NBUF):
        wait_gather(b, b)
        scale(b)
        start_store(b, b)
        start_gather(b + NBUF, b)

    def step(i, _):
        for b in range(NBUF):
            j = i * NBUF + b
            wait_gather(j, b)
            wait_store(j - NBUF, b)
            scale(b)
            start_store(j, b)
            start_gather(j + NBUF, b)
        return 0

    lax.fori_loop(1, NSTEP - 1, step, 0)

    for b in range(NBUF):
        j = (NSTEP - 1) * NBUF + b
        wait_gather(j, b)
        wait_store(j - NBUF, b)
        scale(b)
        start_store(j, b)
    for b in range(NBUF):
        wait_store((NSTEP - 1) * NBUF + b, b)


def kernel(x, table):
    mesh = plsc.VectorSubcoreMesh(core_axis_name="c", subcore_axis_name="s")
    out = pl.kernel(
        _body,
        mesh=mesh,
        out_type=jax.ShapeDtypeStruct((B, S, DIM), jnp.float32),
        scratch_types=[
            pltpu.VMEM((RPW, S), jnp.int32),
            pltpu.VMEM((NBUF, S, DIM), jnp.float32),
            pltpu.VMEM((NBUF, S, DIM), jnp.float32),
            pltpu.SemaphoreType.DMA((NBUF,)),
            pltpu.SemaphoreType.DMA((NBUF,)),
        ],
        compiler_params=pltpu.CompilerParams(use_tc_tiling_on_sc=False),
    )(x.astype(jnp.int32), table)
    return out
